# baseline (speedup 1.0000x reference)
import functools

import jax
import jax.numpy as jnp
from jax import lax
from jax.experimental import pallas as pl
from jax.experimental.pallas import tpu as pltpu
from jax.experimental.pallas import tpu_sc as plsc

_NC = 2
_NS = 16
_NW = _NC * _NS
_L = 16
_BB = 128
_Q = 256


@functools.lru_cache(maxsize=None)
def _make_gather(b, t, vocab):
    nq = 4
    tpad = vocab % 128
    nct = b // _BB
    thalf = t // 2
    nsu = thalf * 2
    mesh = plsc.VectorSubcoreMesh(core_axis_name="c", subcore_axis_name="s")

    @functools.partial(
        pl.kernel,
        mesh=mesh,
        compiler_params=pltpu.CompilerParams(needs_layout_passes=False),
        out_type=jax.ShapeDtypeStruct((t, vocab, b), jnp.float32),
        scratch_types=[
            pltpu.VMEM((32, _BB), jnp.int32),
            [pltpu.VMEM((_BB, 2, 128), jnp.float32) for _ in range(2)],
            [pltpu.VMEM((128, _BB), jnp.float32) for _ in range(2)],
            [pltpu.VMEM((_BB,), jnp.int32) for _ in range(2)],
            [pltpu.SemaphoreType.DMA for _ in range(2)],
            [pltpu.SemaphoreType.DMA for _ in range(2)],
        ],
    )
    def gather(idxc_hbm, tq_hbm, out_hbm, idx_v, inb, outb, idxr, gsem, wsem):
        wid = lax.axis_index("s") * _NC + lax.axis_index("c")
        h = wid & 1
        ct = (wid >> 1) & 7
        tg = wid >> 4
        boff = pl.multiple_of(ct * _BB, 8)
        pltpu.sync_copy(idxc_hbm.at[ct, pl.ds(pl.multiple_of(24 * tg, 8), 32)],
                        idx_v)

        rv = [lax.iota(jnp.int32, _L) + _L * k for k in range(8)]

        def build_idx(su, p):
            tloc = tg + lax.div(su, 2)
            qbase = (2 * h + p) * vocab
            for k in range(8):
                idxr[p][pl.ds(_L * k, _L)] = idx_v[tloc, pl.ds(_L * k, _L)] + qbase

        def issue_gather(p):
            pltpu.async_copy(tq_hbm.at[idxr[p]], inb[p], gsem[p])

        def wait_gather(p):
            pltpu.make_async_copy(tq_hbm.at[idxr[p]], inb[p], gsem[p]).wait()

        def wr_descs(su, p, half, rows):
            tt = tg * thalf + lax.div(su, 2)
            voff = pl.multiple_of(512 * h + _Q * p + 128 * half, 8)
            return (outb[half].at[pl.ds(0, rows)],
                    out_hbm.at[tt, pl.ds(voff, rows), pl.ds(boff, _BB)])

        def issue_write(su, p, half, rows):
            src, dst = wr_descs(su, p, half, rows)
            pltpu.async_copy(src, dst, wsem[half])

        def wait_write(su, p, half, rows):
            src, dst = wr_descs(su, p, half, rows)
            pltpu.make_async_copy(src, dst, wsem[half]).wait()

        def transpose_half(p, half):
            hs = jnp.full((_L,), half, jnp.int32)

            @pl.loop(0, 128)
            def _col(cl):
                cs = jnp.full((_L,), cl, jnp.int32)
                for k in range(8):
                    outb[half][cl, pl.ds(_L * k, _L)] = plsc.load_gather(
                        inb[p], [rv[k], hs, cs])

        def last_rows(p, half):
            return tpad if (p == 1 and half == 1) else 128

        def wr_branch(fn, su, p, half, prev_su=False):
            if last_rows(p, half) == 128:
                fn(su, p, half, 128)
            else:
                @pl.when(h == 0)
                def _():
                    fn(su, p, half, 128)

                @pl.when(h == 1)
                def _():
                    fn(su, p, half, tpad)

        build_idx(0, 0)
        issue_gather(0)
        build_idx(1, 1)
        issue_gather(1)

        @pl.loop(0, nsu, step=2)
        def _body(j0):
            for p in range(2):
                su = j0 + p
                wait_gather(p)
                for half in range(2):
                    @pl.when(su >= 2)
                    def _():
                        wr_branch(wait_write, su - 2, p, half)

                    transpose_half(p, half)
                    wr_branch(issue_write, su, p, half)

                @pl.when(su + 2 < nsu)
                def _():
                    build_idx(su + 2, p)
                    issue_gather(p)

        for p in range(2):
            for half in range(2):
                wr_branch(wait_write, nsu - 2 + p, p, half)

    return gather


def kernel(idx, table):
    b, t = idx.shape
    vocab = table.shape[1]
    vp = (vocab + 127) // 128 * 128
    idxc = (jnp.pad(idx.astype(jnp.int32).T, ((0, -t % 8), (0, 0)))
            .reshape(-1, b // _BB, _BB).transpose(1, 0, 2))
    tq = (jnp.pad(table, ((0, 0), (0, vp - vocab)))
          .reshape(vocab, 4, 2, 128).transpose(1, 0, 2, 3)
          .reshape(4 * vocab, 2, 128))
    out3 = _make_gather(b, t, vocab)(idxc, tq)
    return jnp.transpose(out3, (2, 0, 1))


# speedup vs baseline: 1.7066x; 1.7066x over previous
import functools

import jax
import jax.numpy as jnp
from jax import lax
from jax.experimental import pallas as pl
from jax.experimental.pallas import tpu as pltpu
from jax.experimental.pallas import tpu_sc as plsc

_NC = 2
_NS = 16
_NW = _NC * _NS
_L = 16
_BB = 128
_Q = 256


@functools.lru_cache(maxsize=None)
def _make_gather(b, t, vocab):
    nq = 4
    tpad = vocab % 128
    nct = b // _BB
    thalf = t // 2
    nsu = thalf * 2
    mesh = plsc.VectorSubcoreMesh(core_axis_name="c", subcore_axis_name="s")

    @functools.partial(
        pl.kernel,
        mesh=mesh,
        compiler_params=pltpu.CompilerParams(needs_layout_passes=False),
        out_type=jax.ShapeDtypeStruct((t, vocab, b), jnp.float32),
        scratch_types=[
            pltpu.VMEM((32, _BB), jnp.int32),
            [pltpu.VMEM((_BB, 2, 128), jnp.float32) for _ in range(2)],
            [pltpu.VMEM((128, _BB), jnp.float32) for _ in range(2)],
            [pltpu.VMEM((_BB,), jnp.int32) for _ in range(2)],
            [pltpu.SemaphoreType.DMA for _ in range(2)],
            [pltpu.SemaphoreType.DMA for _ in range(2)],
        ],
    )
    def gather(idxc_hbm, tq_hbm, out_hbm, idx_v, inb, outb, idxr, gsem, wsem):
        wid = lax.axis_index("s") * _NC + lax.axis_index("c")
        h = wid & 1
        ct = (wid >> 1) & 7
        tg = wid >> 4
        boff = pl.multiple_of(ct * _BB, 8)
        pltpu.sync_copy(idxc_hbm.at[ct, pl.ds(pl.multiple_of(24 * tg, 8), 32)],
                        idx_v)

        rv = [lax.iota(jnp.int32, _L) + _L * k for k in range(8)]

        def build_idx(su, p):
            tloc = tg + lax.div(su, 2)
            qbase = (2 * h + p) * vocab
            for k in range(8):
                idxr[p][pl.ds(_L * k, _L)] = idx_v[tloc, pl.ds(_L * k, _L)] + qbase

        def issue_gather(p):
            pltpu.async_copy(tq_hbm.at[idxr[p]], inb[p], gsem[p])

        def wait_gather(p):
            pltpu.make_async_copy(tq_hbm.at[idxr[p]], inb[p], gsem[p]).wait()

        def wr_descs(su, p, half, rows):
            tt = tg * thalf + lax.div(su, 2)
            voff = pl.multiple_of(512 * h + _Q * p + 128 * half, 8)
            return (outb[half].at[pl.ds(0, rows)],
                    out_hbm.at[tt, pl.ds(voff, rows), pl.ds(boff, _BB)])

        def issue_write(su, p, half, rows):
            src, dst = wr_descs(su, p, half, rows)
            pltpu.async_copy(src, dst, wsem[half])

        def wait_write(su, p, half, rows):
            src, dst = wr_descs(su, p, half, rows)
            pltpu.make_async_copy(src, dst, wsem[half]).wait()

        def transpose_half(p, half):
            @plsc.parallel_loop(0, _BB)
            def _row(r):
                rs = jnp.full((_L,), r, jnp.int32)
                for k in range(8):
                    v = inb[p][r, half, pl.ds(_L * k, _L)]
                    plsc.store_scatter(outb[half], [rv[k], rs], v)

        def last_rows(p, half):
            return tpad if (p == 1 and half == 1) else 128

        def wr_branch(fn, su, p, half, prev_su=False):
            if last_rows(p, half) == 128:
                fn(su, p, half, 128)
            else:
                @pl.when(h == 0)
                def _():
                    fn(su, p, half, 128)

                @pl.when(h == 1)
                def _():
                    fn(su, p, half, tpad)

        build_idx(0, 0)
        issue_gather(0)
        build_idx(1, 1)
        issue_gather(1)

        @pl.loop(0, nsu, step=2)
        def _body(j0):
            for p in range(2):
                su = j0 + p
                wait_gather(p)
                for half in range(2):
                    @pl.when(su >= 2)
                    def _():
                        wr_branch(wait_write, su - 2, p, half)

                    transpose_half(p, half)
                    wr_branch(issue_write, su, p, half)

                @pl.when(su + 2 < nsu)
                def _():
                    build_idx(su + 2, p)
                    issue_gather(p)

        for p in range(2):
            for half in range(2):
                wr_branch(wait_write, nsu - 2 + p, p, half)

    return gather


def kernel(idx, table):
    b, t = idx.shape
    vocab = table.shape[1]
    vp = (vocab + 127) // 128 * 128
    idxc = (jnp.pad(idx.astype(jnp.int32).T, ((0, -t % 8), (0, 0)))
            .reshape(-1, b // _BB, _BB).transpose(1, 0, 2))
    tq = (jnp.pad(table, ((0, 0), (0, vp - vocab)))
          .reshape(vocab, 4, 2, 128).transpose(1, 0, 2, 3)
          .reshape(4 * vocab, 2, 128))
    out3 = _make_gather(b, t, vocab)(idxc, tq)
    return jnp.transpose(out3, (2, 0, 1))
